# Initial kernel scaffold; baseline (speedup 1.0000x reference)
import jax
import jax.numpy as jnp
from jax.experimental import pallas as pl


def kernel(features, edge_index, W1, al1, ar1, b1, W2, al2, ar2, b2):
    raise NotImplementedError("write your pallas kernel here")


# speedup vs baseline: 38.4916x; 38.4916x over previous
import functools

import jax
import jax.numpy as jnp
from jax import lax
from jax.experimental import pallas as pl
from jax.experimental.pallas import tpu as pltpu
from jax.experimental.pallas import tpu_sc as plsc

N = 10000
E = 320000
IN_SIZE = 128
HID = 16
OUT_SIZE = 64
HEADS = 8

NC = 2
NS = 16
NW = NC * NS
EPT = E // NW
RB = 624
TAIL0 = NS * RB
TAILN = N - TAIL0
KA = 1000
KB = 200

_f32 = jnp.float32
_i32 = jnp.int32


def _vgather(x, idx):
    return lax.gather(
        x, idx[:, None],
        dimension_numbers=lax.GatherDimensionNumbers(
            offset_dims=(), collapsed_slice_dims=(0,), start_index_map=(0,)),
        slice_sizes=(1,),
        mode=lax.GatherScatterMode.PROMISE_IN_BOUNDS)


def _dense1(x, W1, al1, ar1, S8):
    R = 1000
    grid = N // R

    def body(x_r, w_r, al_r, ar_r, s8_r, h_r, tab_r, gm_r):
        pid = pl.program_id(0)
        h = jnp.dot(x_r[...], w_r[...], preferred_element_type=_f32)
        h_r[...] = h
        el = jnp.dot(h * al_r[...], s8_r[...], preferred_element_type=_f32)
        er = jnp.dot(h * ar_r[...], s8_r[...], preferred_element_type=_f32)
        tab_r[...] = jnp.concatenate([el, er], axis=1)
        m = jnp.max(el, axis=0, keepdims=True)
        rowb = jnp.broadcast_to(jnp.concatenate([m, m], axis=1), (8, 16))

        @pl.when(pid == 0)
        def _():
            gm_r[...] = rowb

        @pl.when(pid != 0)
        def _():
            gm_r[...] = jnp.maximum(gm_r[...], rowb)

    return pl.pallas_call(
        body,
        grid=(grid,),
        in_specs=[
            pl.BlockSpec((R, IN_SIZE), lambda i: (i, 0)),
            pl.BlockSpec((IN_SIZE, IN_SIZE), lambda i: (0, 0)),
            pl.BlockSpec((1, IN_SIZE), lambda i: (0, 0)),
            pl.BlockSpec((1, IN_SIZE), lambda i: (0, 0)),
            pl.BlockSpec((IN_SIZE, 8), lambda i: (0, 0)),
        ],
        out_specs=[
            pl.BlockSpec((R, IN_SIZE), lambda i: (i, 0)),
            pl.BlockSpec((R, 16), lambda i: (i, 0)),
            pl.BlockSpec((8, 16), lambda i: (0, 0)),
        ],
        out_shape=[
            jax.ShapeDtypeStruct((N, IN_SIZE), _f32),
            jax.ShapeDtypeStruct((N, 16), _f32),
            jax.ShapeDtypeStruct((8, 16), _f32),
        ],
    )(x, W1, al1, ar1, S8)


def _pass_a(src, dst, tab, gmax, z16):
    mesh = plsc.VectorSubcoreMesh(core_axis_name="c", subcore_axis_name="s")

    @functools.partial(
        pl.kernel,
        out_type=[jax.ShapeDtypeStruct((E, 16), _f32),
                  jax.ShapeDtypeStruct((NC, N, 16), _f32)],
        mesh=mesh,
        compiler_params=pltpu.CompilerParams(use_tc_tiling_on_sc=False),
        scratch_types=[
            pltpu.VMEM((KA,), _i32), pltpu.VMEM((KA,), _i32),
            pltpu.VMEM((KA, 16), _f32), pltpu.VMEM((KA, 16), _f32),
            pltpu.VMEM((KA, 16), _f32), pltpu.VMEM((16,), _f32),
            pltpu.VMEM_SHARED((N, 16), _f32),
            pltpu.SemaphoreType.DMA, pltpu.SemaphoreType.DMA,
        ],
    )
    def k(src_h, dst_h, tab_h, gm_h, z_h, ehat_h, s_h,
          srcb, dstb, srows, drows, ebuf, gbuf, sacc, sem1, sem2):
        cid = lax.axis_index("c")
        sid = lax.axis_index("s")
        w = sid * NC + cid
        r0 = sid * RB
        pltpu.sync_copy(z_h.at[pl.ds(r0, RB)], sacc.at[pl.ds(r0, RB)])

        @pl.when(sid == 0)
        def _():
            pltpu.sync_copy(z_h.at[pl.ds(TAIL0, TAILN)],
                            sacc.at[pl.ds(TAIL0, TAILN)])

        pltpu.sync_copy(gm_h.at[0], gbuf)
        plsc.subcore_barrier()

        lanes = lax.broadcasted_iota(_i32, (16,), 0)
        sel = lanes < 8
        i07 = lax.bitwise_and(lanes, 7)
        i7p8 = i07 + 8
        gv = gbuf[...]

        def chunk(j, carry):
            off = w * EPT + j * KA
            pltpu.sync_copy(src_h.at[pl.ds(off, KA)], srcb)
            pltpu.sync_copy(dst_h.at[pl.ds(off, KA)], dstb)
            cp1 = pltpu.async_copy(tab_h.at[srcb], srows, sem1)
            cp2 = pltpu.async_copy(tab_h.at[dstb], drows, sem2)
            cp1.wait()
            cp2.wait()

            def pair(i, c2):
                s0 = srows[2 * i]
                s1 = srows[2 * i + 1]
                d0 = drows[2 * i]
                d1 = drows[2 * i + 1]
                el2 = jnp.where(sel, s0, _vgather(s1, i07))
                er2 = jnp.where(sel, _vgather(d0, i7p8), d1)
                z = el2 + er2
                e = jnp.where(z >= 0.0, z, 0.2 * z)
                mh = jnp.maximum(gv + er2, 0.0)
                eh = jnp.exp(e - mh)
                ebuf[2 * i] = jnp.where(sel, eh, _vgather(eh, i07))
                ebuf[2 * i + 1] = jnp.where(sel, _vgather(eh, i7p8), eh)
                return c2

            lax.fori_loop(0, KA // 2, pair, 0, unroll=4)
            pltpu.sync_copy(ebuf, ehat_h.at[pl.ds(off, KA)])
            pltpu.sync_copy(ebuf, sacc.at[dstb], add=True)
            return carry

        lax.fori_loop(0, EPT // KA, chunk, 0)
        plsc.subcore_barrier()
        pltpu.sync_copy(sacc.at[pl.ds(r0, RB)],
                        s_h.at[cid, pl.ds(r0, RB)])

        @pl.when(sid == 0)
        def _():
            pltpu.sync_copy(sacc.at[pl.ds(TAIL0, TAILN)],
                            s_h.at[cid, pl.ds(TAIL0, TAILN)])

    return k(src, dst, tab, gmax, z16)


def _recip(s):
    R = 1000
    grid = N // R

    def body(a_r, b_r, o_r):
        o_r[...] = 1.0 / (a_r[...] + b_r[...] + 1e-9)

    return pl.pallas_call(
        body,
        grid=(grid,),
        in_specs=[pl.BlockSpec((R, 16), lambda i: (i, 0)),
                  pl.BlockSpec((R, 16), lambda i: (i, 0))],
        out_specs=pl.BlockSpec((R, 16), lambda i: (i, 0)),
        out_shape=jax.ShapeDtypeStruct((N, 16), _f32),
    )(s[0], s[1])


def _pass_b(src, dst, ehat, rs, tables, z128, heads_of):
    C = len(tables)
    mesh = plsc.VectorSubcoreMesh(core_axis_name="c", subcore_axis_name="s")

    @functools.partial(
        pl.kernel,
        out_type=[jax.ShapeDtypeStruct((NC, N, 128), _f32) for _ in range(C)],
        mesh=mesh,
        compiler_params=pltpu.CompilerParams(use_tc_tiling_on_sc=False),
        scratch_types=[
            pltpu.VMEM((KB,), _i32), pltpu.VMEM((KB,), _i32),
            pltpu.VMEM((KB, 128), _f32), pltpu.VMEM((KB, 16), _f32),
            pltpu.VMEM((KB, 16), _f32),
            pltpu.VMEM_SHARED((N, 128), _f32),
            pltpu.SemaphoreType.DMA, pltpu.SemaphoreType.DMA,
        ],
    )
    def k(src_h, dst_h, ehat_h, rs_h, *rest):
        tabs = rest[:C]
        z_h = rest[C]
        outs = rest[C + 1:2 * C + 1]
        (srcb, dstb, hrows, ebuf, rsrows, acc, sem1, sem2) = rest[2 * C + 1:]
        cid = lax.axis_index("c")
        sid = lax.axis_index("s")
        w = sid * NC + cid
        r0 = sid * RB
        splats = [jnp.full((16,), h, _i32) for h in range(HEADS)]

        for c in range(C):
            pltpu.sync_copy(z_h.at[pl.ds(r0, RB)], acc.at[pl.ds(r0, RB)])

            @pl.when(sid == 0)
            def _():
                pltpu.sync_copy(z_h.at[pl.ds(TAIL0, TAILN)],
                                acc.at[pl.ds(TAIL0, TAILN)])

            plsc.subcore_barrier()

            def echunk(j, carry, _c=c):
                off = w * EPT + j * KB
                pltpu.sync_copy(src_h.at[pl.ds(off, KB)], srcb)
                pltpu.sync_copy(dst_h.at[pl.ds(off, KB)], dstb)
                cph = pltpu.async_copy(tabs[_c].at[srcb], hrows, sem1)
                cpr = pltpu.async_copy(rs_h.at[dstb], rsrows, sem2)
                pltpu.sync_copy(ehat_h.at[pl.ds(off, KB)], ebuf)
                cph.wait()
                cpr.wait()

                def edge(kk, c2):
                    arow = ebuf[kk] * rsrows[kk]
                    for jj in range(8):
                        sp = _vgather(arow, splats[heads_of[_c][jj]])
                        hrows[kk, pl.ds(16 * jj, 16)] = (
                            hrows[kk, pl.ds(16 * jj, 16)] * sp)
                    return c2

                lax.fori_loop(0, KB, edge, 0, unroll=4)
                pltpu.sync_copy(hrows, acc.at[dstb], add=True)
                return carry

            lax.fori_loop(0, EPT // KB, echunk, 0)
            plsc.subcore_barrier()
            pltpu.sync_copy(acc.at[pl.ds(r0, RB)],
                            outs[c].at[cid, pl.ds(r0, RB)])

            @pl.when(sid == 0)
            def _():
                pltpu.sync_copy(acc.at[pl.ds(TAIL0, TAILN)],
                                outs[c].at[cid, pl.ds(TAIL0, TAILN)])

    return k(src, dst, ehat, rs, *tables, z128)


def _dense2(p0, p1, b1, W2, al2, ar2, S8):
    R = 1000
    grid = N // R

    def body(p0_r, p1_r, b1_r, w2_r, al_r, ar_r, s8_r,
             h0_r, h1_r, h2_r, h3_r, tab_r, gm_r):
        pid = pl.program_id(0)
        o1 = jnp.maximum(p0_r[...] + p1_r[...] + b1_r[...], 0.0)
        h2 = jnp.dot(o1, w2_r[...], preferred_element_type=_f32)
        h0_r[...] = h2[:, 0:128]
        h1_r[...] = h2[:, 128:256]
        h2_r[...] = h2[:, 256:384]
        h3_r[...] = h2[:, 384:512]
        el = jnp.dot(h2 * al_r[...], s8_r[...], preferred_element_type=_f32)
        er = jnp.dot(h2 * ar_r[...], s8_r[...], preferred_element_type=_f32)
        tab_r[...] = jnp.concatenate([el, er], axis=1)
        m = jnp.max(el, axis=0, keepdims=True)
        rowb = jnp.broadcast_to(jnp.concatenate([m, m], axis=1), (8, 16))

        @pl.when(pid == 0)
        def _():
            gm_r[...] = rowb

        @pl.when(pid != 0)
        def _():
            gm_r[...] = jnp.maximum(gm_r[...], rowb)

    F = HEADS * OUT_SIZE
    return pl.pallas_call(
        body,
        grid=(grid,),
        in_specs=[
            pl.BlockSpec((R, 128), lambda i: (i, 0)),
            pl.BlockSpec((R, 128), lambda i: (i, 0)),
            pl.BlockSpec((1, 128), lambda i: (0, 0)),
            pl.BlockSpec((128, F), lambda i: (0, 0)),
            pl.BlockSpec((1, F), lambda i: (0, 0)),
            pl.BlockSpec((1, F), lambda i: (0, 0)),
            pl.BlockSpec((F, 8), lambda i: (0, 0)),
        ],
        out_specs=[pl.BlockSpec((R, 128), lambda i: (i, 0)) for _ in range(4)]
        + [pl.BlockSpec((R, 16), lambda i: (i, 0)),
           pl.BlockSpec((8, 16), lambda i: (0, 0))],
        out_shape=[jax.ShapeDtypeStruct((N, 128), _f32) for _ in range(4)]
        + [jax.ShapeDtypeStruct((N, 16), _f32),
           jax.ShapeDtypeStruct((8, 16), _f32)],
    )(p0, p1, b1, W2, al2, ar2, S8)


def _final(parts, b2c, M128):
    R = 1000
    grid = N // R

    def body(q00, q01, q10, q11, q20, q21, q30, q31, bc_r, m_r, o_r):
        qs = [(q00, q01), (q10, q11), (q20, q21), (q30, q31)]
        z = jnp.zeros((R, OUT_SIZE), _f32)
        for c in range(4):
            a, b = qs[c]
            t = a[...] + b[...] + bc_r[pl.ds(c, 1), :]
            z = z + jnp.dot(t, m_r[...], preferred_element_type=_f32)
        t = z - jnp.max(z, axis=1, keepdims=True)
        o_r[...] = t - jnp.log(jnp.sum(jnp.exp(t), axis=1, keepdims=True))

    return pl.pallas_call(
        body,
        grid=(grid,),
        in_specs=[pl.BlockSpec((R, 128), lambda i: (i, 0))
                  for _ in range(8)]
        + [pl.BlockSpec((4, 128), lambda i: (0, 0)),
           pl.BlockSpec((128, OUT_SIZE), lambda i: (0, 0))],
        out_specs=pl.BlockSpec((R, OUT_SIZE), lambda i: (i, 0)),
        out_shape=jax.ShapeDtypeStruct((N, OUT_SIZE), _f32),
    )(*parts, b2c, M128)


def kernel(features, edge_index, W1, al1, ar1, b1, W2, al2, ar2, b2):
    src = edge_index[0].astype(_i32)
    dst = edge_index[1].astype(_i32)

    al1r = al1.reshape(1, HEADS * HID)
    ar1r = ar1.reshape(1, HEADS * HID)
    al2r = al2.reshape(1, HEADS * OUT_SIZE)
    ar2r = ar2.reshape(1, HEADS * OUT_SIZE)
    b1r = b1.reshape(1, HEADS * HID)
    b2c = b2.reshape(4, 128)
    hid_sel = jnp.equal(
        jnp.arange(HEADS * HID)[:, None] // HID,
        jnp.arange(HEADS)[None, :]).astype(_f32)
    out_sel = jnp.equal(
        jnp.arange(HEADS * OUT_SIZE)[:, None] // OUT_SIZE,
        jnp.arange(HEADS)[None, :]).astype(_f32)
    mean_m = jnp.tile(jnp.eye(OUT_SIZE, dtype=_f32), (2, 1)) / HEADS
    z16 = jnp.zeros((N, 16), _f32)
    z128 = jnp.zeros((N, 128), _f32)

    h1, tab1, gm1 = _dense1(features, W1, al1r, ar1r, hid_sel)
    ehat1, s1 = _pass_a(src, dst, tab1, gm1, z16)
    rs1 = _recip(s1)
    (p1,) = _pass_b(src, dst, ehat1, rs1, [h1], z128,
                    heads_of=[list(range(8))])

    h2c = _dense2(p1[0], p1[1], b1r, W2, al2r, ar2r, out_sel)
    h2tabs, tab2, gm2 = list(h2c[:4]), h2c[4], h2c[5]
    ehat2, s2 = _pass_a(src, dst, tab2, gm2, z16)
    rs2 = _recip(s2)
    heads_of2 = [[2 * c + jj // 4 for jj in range(8)] for c in range(4)]
    p2 = _pass_b(src, dst, ehat2, rs2, h2tabs, z128, heads_of=heads_of2)

    parts = []
    for c in range(4):
        parts.append(p2[c][0])
        parts.append(p2[c][1])
    return _final(parts, b2c, mean_m)
